# baseline (speedup 1.0000x reference)
import numpy as np
import jax
import jax.numpy as jnp
from jax.experimental import pallas as pl
from jax.experimental.pallas import tpu as pltpu

_N = 100000
_V = 512
_D = 128
_SQD = np.float32(np.sqrt(_D))
_SQAVG = np.float32(np.sqrt(_N / _V))
_SQ2 = np.float32(np.sqrt(2.0))


def _silu_cst():
    z = np.random.RandomState(0).randn(1_000_000).astype(np.float64)
    s = z / (1.0 + np.exp(-z))
    return np.float32(1.0 / np.sqrt(np.mean(s * s)))


_CST = _silu_cst()

_B = 2000
_NB = _N // _B


def _body(x_ref, pos_ref, b_ref, xv_ref, wvsc_ref, wtp_ref, wn2v_ref,
          wnsc_ref, xvout_ref, out_ref, snode_ref, seg_ref, tab_ref):
    p = pl.program_id(0)
    i = pl.program_id(1)
    b = b_ref[0]
    iota = jax.lax.broadcasted_iota(jnp.int32, (_V, _B), 0)

    @pl.when(p == 0)
    def _phase_a():
        x = x_ref[...]
        ohT = jnp.where(iota == b, jnp.float32(1.0), jnp.float32(0.0))
        y = x * pos_ref[...]
        part = jax.lax.dot_general(ohT, y, (((1,), (0,)), ((), ())),
                                   preferred_element_type=jnp.float32)

        @pl.when(i == 0)
        def _():
            seg_ref[...] = part

        @pl.when(i > 0)
        def _():
            seg_ref[...] += part

        snode_ref[pl.ds(i * _B, _B), :] = jnp.dot(
            x, wnsc_ref[...], preferred_element_type=jnp.float32) / _SQD

    @pl.when((p == 1) & (i == 0))
    def _virtual():
        s_virtual = jnp.dot(xv_ref[...], wvsc_ref[...],
                            preferred_element_type=jnp.float32) / _SQD
        m = jnp.dot(seg_ref[...], wtp_ref[...],
                    preferred_element_type=jnp.float32) / (_SQD * _SQAVG)
        m = _CST * m * jax.nn.sigmoid(m)
        xv_out = (s_virtual + m) / _SQ2
        xvout_ref[...] = xv_out
        t = jnp.dot(xv_out, wn2v_ref[...],
                    preferred_element_type=jnp.float32) / _SQD
        tab_ref[...] = _CST * t * jax.nn.sigmoid(t)

    @pl.when(p == 1)
    def _phase_b():
        ohT = jnp.where(iota == b, jnp.float32(1.0), jnp.float32(0.0))
        gathered = jax.lax.dot_general(
            ohT, tab_ref[...], (((0,), (0,)), ((), ())),
            preferred_element_type=jnp.float32)
        out_ref[...] = (snode_ref[pl.ds(i * _B, _B), :] + gathered) / _SQ2


def kernel(x_virtual, x_node, node_pos_sh, batch, W_vsc, W_nsc, W_tp, W_n2v):
    b3 = batch.astype(jnp.int32).reshape(_NB, 1, _B)

    xv_out, x_node_out = pl.pallas_call(
        _body,
        grid=(2, _NB),
        in_specs=[
            pl.BlockSpec((_B, _D), lambda p, i: ((1 - p) * i, 0)),
            pl.BlockSpec((_B, 1), lambda p, i: ((1 - p) * i, 0)),
            pl.BlockSpec((1, 1, _B), lambda p, i: (i, 0, 0)),
            pl.BlockSpec((_V, _D), lambda p, i: (0, 0)),
            pl.BlockSpec((_D, _D), lambda p, i: (0, 0)),
            pl.BlockSpec((_D, _D), lambda p, i: (0, 0)),
            pl.BlockSpec((_D, _D), lambda p, i: (0, 0)),
            pl.BlockSpec((_D, _D), lambda p, i: (0, 0)),
        ],
        out_specs=(
            pl.BlockSpec((_V, _D), lambda p, i: (0, 0)),
            pl.BlockSpec((_B, _D), lambda p, i: (p * i, 0)),
        ),
        out_shape=(
            jax.ShapeDtypeStruct((_V, _D), jnp.float32),
            jax.ShapeDtypeStruct((_N, _D), jnp.float32),
        ),
        scratch_shapes=[
            pltpu.VMEM((_N, _D), jnp.float32),
            pltpu.VMEM((_V, _D), jnp.float32),
            pltpu.VMEM((_V, _D), jnp.float32),
        ],
    )(x_node, node_pos_sh, b3, x_virtual, W_vsc, W_tp, W_n2v, W_nsc)

    return (xv_out, x_node_out)
